# baseline (speedup 1.0000x reference)
import jax
import jax.numpy as jnp
from jax.experimental import pallas as pl
from jax.experimental.pallas import tpu as pltpu

N = 35
HID = 120
H = 16
_NEG = -1e30
_NS = 15
_SW = 128


def _gat_kernel(adj_ref, W1_ref, as1_ref, ad1_ref, b1_ref, W2_hbm,
                as2_ref, ad2_ref, b2_ref, W3_ref, as3_ref, ad3_ref, b3_ref,
                Wfc_ref, bfc_ref, out_ref, w2_vmem, sems):
    f32 = jnp.float32

    def slab_copy(q):
        return pltpu.make_async_copy(
            W2_hbm.at[:, pl.ds(q * _SW, _SW)],
            w2_vmem.at[:, pl.ds(q * _SW, _SW)],
            sems.at[q])

    for q in range(_NS):
        slab_copy(q).start()

    adj = adj_ref[:]
    ii = jax.lax.broadcasted_iota(jnp.int32, (N, N), 0)
    jj = jax.lax.broadcasted_iota(jnp.int32, (N, N), 1)
    countf = (adj > 0.5).astype(f32) + (ii == jj).astype(f32)
    has_edge = countf > 0.0

    def head_attn(hs, asr, adr):
        col = jax.lax.dot_general(
            hs, asr, (((1,), (1,)), ((), ())), preferred_element_type=f32)
        row = jax.lax.dot_general(
            adr, hs, (((1,), (1,)), ((), ())), preferred_element_type=f32)
        e = col + row
        e = jnp.where(e >= 0.0, e, 0.2 * e)
        e = jnp.where(has_edge, e, _NEG)
        m = jnp.max(e, axis=0, keepdims=True)
        ex = jnp.exp(e - m) * countf
        s = jnp.sum(ex, axis=0, keepdims=True)
        p = ex / (s + 1e-16)
        return jax.lax.dot_general(
            p, hs, (((0,), (0,)), ((), ())), preferred_element_type=f32)

    def heads_block(h, a_src, a_dst, head_ids, C):
        return [head_attn(h[:, k * C:(k + 1) * C],
                          a_src[hd:hd + 1, :], a_dst[hd:hd + 1, :])
                for k, hd in enumerate(head_ids)]

    def elu(x):
        return jnp.where(x > 0.0, x, jnp.exp(jnp.minimum(x, 0.0)) - 1.0)

    h1 = jnp.dot(adj, W1_ref[:], preferred_element_type=f32)
    o1 = heads_block(h1, as1_ref[:], ad1_ref[:], list(range(H)), HID)
    x1 = elu(jnp.concatenate(o1, axis=1) + jnp.reshape(b1_ref[:], (1, H * HID)))

    as2 = as2_ref[:]
    ad2 = ad2_ref[:]
    blocks = []
    o2 = []
    k = 0
    for q in range(_NS):
        slab_copy(q).wait()
        blocks.append(jnp.dot(x1, w2_vmem[:, q * _SW:(q + 1) * _SW],
                              preferred_element_type=f32))
        while k < H and (HID * (k + 1) - 1) // _SW <= q:
            b0 = (HID * k) // _SW
            off = HID * k - _SW * b0
            if off + HID <= _SW:
                hk = blocks[b0][:, off:off + HID]
            else:
                hk = jnp.concatenate(
                    blocks[b0:b0 + 2], axis=1)[:, off:off + HID]
            o2.append(head_attn(hk, as2[k:k + 1, :], ad2[k:k + 1, :]))
            k += 1
    x2 = elu(jnp.concatenate(o2, axis=1) + jnp.reshape(b2_ref[:], (1, H * HID)))

    h3 = jnp.dot(x2, W3_ref[:], preferred_element_type=f32)
    o3 = head_attn(h3, as3_ref[:], ad3_ref[:])
    x3 = o3 + jnp.reshape(b3_ref[:], (1, HID))
    out = (jnp.dot(x3, Wfc_ref[:], preferred_element_type=f32)
           + jnp.reshape(bfc_ref[:], (1, N)))
    out_ref[:] = jnp.maximum(out, 0.0)


def _full(shape):
    nd = len(shape)
    return pl.BlockSpec(shape, lambda i: (0,) * nd)


def kernel(adj_matrix, W1, as1, ad1, b1, W2, as2, ad2, b2,
           W3, as3, ad3, b3, Wfc, bfc):
    KC = H * HID
    in_specs = [
        _full((N, N)), _full((N, KC)), _full((H, HID)), _full((H, HID)),
        _full((KC,)),
        pl.BlockSpec(memory_space=pltpu.MemorySpace.HBM),
        _full((H, HID)), _full((H, HID)), _full((KC,)),
        _full((KC, HID)), _full((1, HID)), _full((1, HID)), _full((HID,)),
        _full((HID, N)), _full((N,)),
    ]
    return pl.pallas_call(
        _gat_kernel,
        out_shape=jax.ShapeDtypeStruct((N, N), jnp.float32),
        grid=(1,),
        in_specs=in_specs,
        out_specs=_full((N, N)),
        scratch_shapes=[
            pltpu.VMEM((KC, KC), jnp.float32),
            pltpu.SemaphoreType.DMA((_NS,)),
        ],
    )(adj_matrix, W1, as1, ad1, b1, W2, as2, ad2, b2,
      W3, as3, ad3, b3, Wfc, bfc)


# speedup vs baseline: 1.3394x; 1.3394x over previous
import jax
import jax.numpy as jnp
from jax.experimental import pallas as pl
from jax.experimental.pallas import tpu as pltpu

N = 35
HID = 120
H = 16
_NEG = -1e30
_NS = 15
_SW = 128


def _gat_kernel(adj_ref, W1_ref, as1_ref, ad1_ref, b1_ref, W2_hbm,
                as2_ref, ad2_ref, b2_ref, W3_ref, as3_ref, ad3_ref, b3_ref,
                Wfc_ref, bfc_ref, out_ref, w2_vmem_a, w2_vmem_b, sems):
    f32 = jnp.float32

    def slab_copy(q):
        dst = w2_vmem_a if q % 2 == 0 else w2_vmem_b
        return pltpu.make_async_copy(
            W2_hbm.at[pl.ds(q * _SW, _SW), :],
            dst.at[pl.ds((q // 2) * _SW, _SW), :],
            sems.at[q])

    def slab_view(q):
        dst = w2_vmem_a if q % 2 == 0 else w2_vmem_b
        return dst[(q // 2) * _SW:(q // 2) * _SW + _SW, :]

    for q in range(_NS):
        slab_copy(q).start()

    adj = adj_ref[:]
    ii = jax.lax.broadcasted_iota(jnp.int32, (N, N), 0)
    jj = jax.lax.broadcasted_iota(jnp.int32, (N, N), 1)
    countf = (adj > 0.5).astype(f32) + (ii == jj).astype(f32)
    has_edge = countf > 0.0

    def heads_block(h, a_src, a_dst, head_ids, C):
        outs = []
        for k, hd in enumerate(head_ids):
            hs = h[:, k * C:(k + 1) * C]
            asr = a_src[hd:hd + 1, :]
            adr = a_dst[hd:hd + 1, :]
            col = jax.lax.dot_general(
                hs, asr, (((1,), (1,)), ((), ())), preferred_element_type=f32)
            row = jax.lax.dot_general(
                adr, hs, (((1,), (1,)), ((), ())), preferred_element_type=f32)
            e = col + row
            e = jnp.where(e >= 0.0, e, 0.2 * e)
            e = jnp.where(has_edge, e, _NEG)
            m = jnp.max(e, axis=0, keepdims=True)
            ex = jnp.exp(e - m) * countf
            s = jnp.sum(ex, axis=0, keepdims=True)
            p = ex / (s + 1e-16)
            outs.append(jax.lax.dot_general(
                p, hs, (((0,), (0,)), ((), ())), preferred_element_type=f32))
        return outs

    def elu(x):
        return jnp.where(x > 0.0, x, jnp.exp(jnp.minimum(x, 0.0)) - 1.0)

    h1 = jnp.dot(adj, W1_ref[:], preferred_element_type=f32)
    o1 = heads_block(h1, as1_ref[:], ad1_ref[:], list(range(H)), HID)
    x1 = elu(jnp.concatenate(o1, axis=1) + jnp.reshape(b1_ref[:], (1, H * HID)))

    h2 = None
    for q in range(_NS):
        slab_copy(q).wait()
        part = jnp.dot(x1[:, q * _SW:(q + 1) * _SW],
                       slab_view(q),
                       preferred_element_type=f32)
        h2 = part if h2 is None else h2 + part
    o2 = heads_block(h2, as2_ref[:], ad2_ref[:], list(range(H)), HID)
    x2 = elu(jnp.concatenate(o2, axis=1) + jnp.reshape(b2_ref[:], (1, H * HID)))

    h3 = jnp.dot(x2, W3_ref[:], preferred_element_type=f32)
    o3 = heads_block(h3, as3_ref[:], ad3_ref[:], [0], HID)[0]
    x3 = o3 + jnp.reshape(b3_ref[:], (1, HID))
    out = (jnp.dot(x3, Wfc_ref[:], preferred_element_type=f32)
           + jnp.reshape(bfc_ref[:], (1, N)))
    out_ref[:] = jnp.maximum(out, 0.0)


def _full(shape):
    nd = len(shape)
    return pl.BlockSpec(shape, lambda i: (0,) * nd)


def kernel(adj_matrix, W1, as1, ad1, b1, W2, as2, ad2, b2,
           W3, as3, ad3, b3, Wfc, bfc):
    KC = H * HID
    in_specs = [
        _full((N, N)), _full((N, KC)), _full((H, HID)), _full((H, HID)),
        _full((KC,)),
        pl.BlockSpec(memory_space=pltpu.MemorySpace.HBM),
        _full((H, HID)), _full((H, HID)), _full((KC,)),
        _full((KC, HID)), _full((1, HID)), _full((1, HID)), _full((HID,)),
        _full((HID, N)), _full((N,)),
    ]
    return pl.pallas_call(
        _gat_kernel,
        out_shape=jax.ShapeDtypeStruct((N, N), jnp.float32),
        grid=(1,),
        in_specs=in_specs,
        out_specs=_full((N, N)),
        scratch_shapes=[
            pltpu.VMEM((8 * _SW, KC), jnp.float32),
            pltpu.VMEM((7 * _SW, KC), jnp.float32),
            pltpu.SemaphoreType.DMA((_NS,)),
        ],
    )(adj_matrix, W1, as1, ad1, b1, W2, as2, ad2, b2,
      W3, as3, ad3, b3, Wfc, bfc)


# speedup vs baseline: 1.3426x; 1.0024x over previous
import jax
import jax.numpy as jnp
from jax.experimental import pallas as pl
from jax.experimental.pallas import tpu as pltpu

N = 35
HID = 120
H = 16
_NEG = -1e30
_NS = 15
_SW = 128


def _gat_kernel(adj_ref, W1_ref, as1_ref, ad1_ref, b1_ref, W2_hbm,
                as2_ref, ad2_ref, b2_ref, W3_ref, as3_ref, ad3_ref, b3_ref,
                Wfc_ref, bfc_ref, out_ref, w2_vmem, sems):
    f32 = jnp.float32

    def slab_copy(q):
        return pltpu.make_async_copy(
            W2_hbm.at[pl.ds(q * _SW, _SW), :],
            w2_vmem.at[pl.ds(q * _SW, _SW), :],
            sems.at[q])

    for q in range(_NS):
        slab_copy(q).start()

    adj = adj_ref[:]
    ii = jax.lax.broadcasted_iota(jnp.int32, (N, N), 0)
    jj = jax.lax.broadcasted_iota(jnp.int32, (N, N), 1)
    countf = (adj > 0.5).astype(f32) + (ii == jj).astype(f32)
    has_edge = countf > 0.0

    def heads_block(h, a_src, a_dst, head_ids, C):
        outs = []
        for k, hd in enumerate(head_ids):
            hs = h[:, k * C:(k + 1) * C]
            asr = a_src[hd:hd + 1, :]
            adr = a_dst[hd:hd + 1, :]
            col = jax.lax.dot_general(
                hs, asr, (((1,), (1,)), ((), ())), preferred_element_type=f32)
            row = jax.lax.dot_general(
                adr, hs, (((1,), (1,)), ((), ())), preferred_element_type=f32)
            e = col + row
            e = jnp.where(e >= 0.0, e, 0.2 * e)
            e = jnp.where(has_edge, e, _NEG)
            m = jnp.max(e, axis=0, keepdims=True)
            ex = jnp.exp(e - m) * countf
            s = jnp.sum(ex, axis=0, keepdims=True)
            p = ex / (s + 1e-16)
            outs.append(jax.lax.dot_general(
                p, hs, (((0,), (0,)), ((), ())), preferred_element_type=f32))
        return outs

    def elu(x):
        return jnp.where(x > 0.0, x, jnp.exp(jnp.minimum(x, 0.0)) - 1.0)

    h1 = jnp.dot(adj, W1_ref[:], preferred_element_type=f32)
    o1 = heads_block(h1, as1_ref[:], ad1_ref[:], list(range(H)), HID)
    x1 = elu(jnp.concatenate(o1, axis=1) + jnp.reshape(b1_ref[:], (1, H * HID)))

    h2 = None
    for q in range(_NS):
        slab_copy(q).wait()
        part = jnp.dot(x1[:, q * _SW:(q + 1) * _SW],
                       w2_vmem[q * _SW:(q + 1) * _SW, :],
                       preferred_element_type=f32)
        h2 = part if h2 is None else h2 + part
    o2 = heads_block(h2, as2_ref[:], ad2_ref[:], list(range(H)), HID)
    x2 = elu(jnp.concatenate(o2, axis=1) + jnp.reshape(b2_ref[:], (1, H * HID)))

    h3 = jnp.dot(x2, W3_ref[:], preferred_element_type=f32)
    o3 = heads_block(h3, as3_ref[:], ad3_ref[:], [0], HID)[0]
    x3 = o3 + jnp.reshape(b3_ref[:], (1, HID))
    out = (jnp.dot(x3, Wfc_ref[:], preferred_element_type=f32)
           + jnp.reshape(bfc_ref[:], (1, N)))
    out_ref[:] = jnp.maximum(out, 0.0)


def _full(shape):
    nd = len(shape)
    return pl.BlockSpec(shape, lambda i: (0,) * nd)


def kernel(adj_matrix, W1, as1, ad1, b1, W2, as2, ad2, b2,
           W3, as3, ad3, b3, Wfc, bfc):
    KC = H * HID
    in_specs = [
        _full((N, N)), _full((N, KC)), _full((H, HID)), _full((H, HID)),
        _full((KC,)),
        pl.BlockSpec(memory_space=pltpu.MemorySpace.HBM),
        _full((H, HID)), _full((H, HID)), _full((KC,)),
        _full((KC, HID)), _full((1, HID)), _full((1, HID)), _full((HID,)),
        _full((HID, N)), _full((N,)),
    ]
    return pl.pallas_call(
        _gat_kernel,
        out_shape=jax.ShapeDtypeStruct((N, N), jnp.float32),
        grid=(1,),
        in_specs=in_specs,
        out_specs=_full((N, N)),
        scratch_shapes=[
            pltpu.VMEM((KC, KC), jnp.float32),
            pltpu.SemaphoreType.DMA((_NS,)),
        ],
    )(adj_matrix, W1, as1, ad1, b1, W2, as2, ad2, b2,
      W3, as3, ad3, b3, Wfc, bfc)
